# baseline (speedup 1.0000x reference)
import functools

import jax
import jax.numpy as jnp
from jax import lax
from jax.experimental import pallas as pl
from jax.experimental.pallas import tpu as pltpu
from jax.experimental.pallas import tpu_sc as plsc

N = 10000
NP = 10112
R2 = 460
R2P = 512
H = 128
E = 320000
T = 3
SLOPE = (1.0 / 8.0 + 1.0 / 3.0) / 2.0

NC = 2
NS = 16
NW = NC * NS
GEOM = {
    "single": dict(ch=128, cpt=80, cph=16, nbuf=2),
    "dual": dict(ch=128, cpt=80, cph=16, nbuf=2),
}
RPT = NP // NS
BPT = R2P // NS

def _fill(ref, nrows, ncols, val):
    v = jnp.full((16,), val, jnp.float32)

    def body(i, _):
        for j in range(ncols // 16):
            ref[i, pl.ds(j * 16, 16)] = v
        return 0

    lax.fori_loop(0, nrows, body, 0)


@functools.cache
def _sc_pass(mode):
    g = GEOM[mode]
    CH, CPT, CPH, NBUF = g["ch"], g["cpt"], g["cph"], g["nbuf"]
    NPH = CPT // CPH
    RF, RR = RPT // CH, RPT % CH
    mesh = plsc.VectorSubcoreMesh(core_axis_name="c", subcore_axis_name="s")
    outs = [jax.ShapeDtypeStruct((NC, NP, H), jnp.float32)]
    scratch = [pltpu.VMEM_SHARED((NP, H), jnp.float32)]
    if mode == "dual":
        outs.append(jax.ShapeDtypeStruct((NC, R2P, H), jnp.float32))
        scratch.append(pltpu.VMEM_SHARED((R2P, H), jnp.float32))
    scratch.append(pltpu.VMEM((CPH, CH), jnp.int32))
    scratch.append(pltpu.VMEM((CPH, CH), jnp.int32))
    if mode == "dual":
        scratch.append(pltpu.VMEM((CPH, CH), jnp.int32))
    scratch.append(pltpu.VMEM((NBUF, CH, H), jnp.float32))
    scratch.extend([pltpu.SemaphoreType.DMA] * NBUF)
    scratch.extend([pltpu.SemaphoreType.DMA] * NBUF)
    if mode == "dual":
        scratch.extend([pltpu.SemaphoreType.DMA] * NBUF)

    def body(*refs):
        it = iter(refs)
        tab = next(it)
        g2 = next(it)
        a2 = next(it)
        b2 = next(it) if mode == "dual" else None
        out_a = next(it)
        out_b = next(it) if mode == "dual" else None
        acc_a = next(it)
        acc_b = next(it) if mode == "dual" else None
        gv = next(it)
        av = next(it)
        bv = next(it) if mode == "dual" else None
        rows = next(it)
        gsems = [next(it) for _ in range(NBUF)]
        asems = [next(it) for _ in range(NBUF)]
        bsems = [next(it) for _ in range(NBUF)] if mode == "dual" else None

        c = lax.axis_index("c")
        s = lax.axis_index("s")
        w = c * NS + s
        lo = w * CPT

        _fill(rows.at[0], CH, H, 0.0)
        for k in range(RF):
            pltpu.sync_copy(rows.at[0],
                            acc_a.at[pl.ds(s * RPT + k * CH, CH)])
        pltpu.sync_copy(rows.at[0, pl.ds(0, RR)],
                        acc_a.at[pl.ds(s * RPT + RF * CH, RR)])
        if mode == "dual":
            pltpu.sync_copy(rows.at[0, pl.ds(0, BPT)],
                            acc_b.at[pl.ds(s * BPT, BPT)])
        plsc.subcore_barrier()

        def phase(ph, _):
            base = lo + ph * CPH
            pltpu.sync_copy(g2.at[pl.ds(base, CPH)], gv)
            pltpu.sync_copy(a2.at[pl.ds(base, CPH)], av)
            if mode == "dual":
                pltpu.sync_copy(b2.at[pl.ds(base, CPH)], bv)
            cps = [None] * NBUF
            scs = [None] * NBUF
            bcs = [None] * NBUF
            cps[0] = pltpu.async_copy(tab.at[gv.at[0]], rows.at[0], gsems[0])
            cps[1] = pltpu.async_copy(tab.at[gv.at[1]], rows.at[1], gsems[1])
            for j in range(CPH):
                b = j % NBUF
                cps[b].wait()
                rb = rows.at[b]
                scs[b] = pltpu.async_copy(rb, acc_a.at[av.at[j]], asems[b],
                                          add=True)
                if mode == "dual":
                    bcs[b] = pltpu.async_copy(rb, acc_b.at[bv.at[j]],
                                              bsems[b], add=True)
                k = j + 2
                if k < CPH:
                    bk = k % NBUF
                    if scs[bk] is not None:
                        scs[bk].wait()
                        if mode == "dual":
                            bcs[bk].wait()
                    cps[bk] = pltpu.async_copy(tab.at[gv.at[k]], rows.at[bk],
                                               gsems[bk])
            for j in range(max(0, CPH - NBUF), CPH):
                b = j % NBUF
                scs[b].wait()
                if mode == "dual":
                    bcs[b].wait()
            return 0

        lax.fori_loop(0, NPH, phase, 0)
        plsc.subcore_barrier()

        for k in range(RF):
            sl = pl.ds(s * RPT + k * CH, CH)
            pltpu.sync_copy(acc_a.at[sl], out_a.at[c, sl])
        sl = pl.ds(s * RPT + RF * CH, RR)
        pltpu.sync_copy(acc_a.at[sl], out_a.at[c, sl])
        if mode == "dual":
            sl = pl.ds(s * BPT, BPT)
            pltpu.sync_copy(acc_b.at[sl], out_b.at[c, sl])

    out_type = outs[0] if len(outs) == 1 else tuple(outs)
    return pl.kernel(body, out_type=out_type, mesh=mesh,
                     scratch_types=tuple(scratch))


def _norm_rows(x):
    ss = jnp.sum(x * x, axis=-1, keepdims=True)
    return x / jnp.maximum(jnp.sqrt(ss), 1e-12)


BN = 1000


def _tc_prep(x):
    def body(x_ref, o_ref):
        o_ref[...] = _norm_rows(x_ref[...])

    return pl.pallas_call(
        body,
        grid=(N // BN,),
        in_specs=[pl.BlockSpec((BN, H), lambda i: (i, 0))],
        out_specs=pl.BlockSpec((BN, H), lambda i: (i, 0)),
        out_shape=jax.ShapeDtypeStruct((N, H), jnp.float32),
    )(x)


def _tc_layer1(aggp, bp, h, wn, wl):
    def body(a_ref, b_ref, h_ref, wn_ref, wl_ref, o_ref):
        a = a_ref[0] + a_ref[1] + b_ref[0] + b_ref[1]
        pre = (jnp.dot(a, wn_ref[...], preferred_element_type=jnp.float32)
               + jnp.dot(h_ref[...], wl_ref[...],
                         preferred_element_type=jnp.float32))
        o_ref[...] = jnp.where(pre >= 0, pre, SLOPE * pre)

    return pl.pallas_call(
        body,
        grid=(N // BN,),
        in_specs=[
            pl.BlockSpec((NC, BN, H), lambda i: (0, i, 0)),
            pl.BlockSpec((NC, BN, H), lambda i: (0, i, 0)),
            pl.BlockSpec((BN, H), lambda i: (i, 0)),
            pl.BlockSpec((H, H), lambda i: (0, 0)),
            pl.BlockSpec((H, H), lambda i: (0, 0)),
        ],
        out_specs=pl.BlockSpec((BN, H), lambda i: (i, 0)),
        out_shape=jax.ShapeDtypeStruct((N, H), jnp.float32),
    )(aggp, bp, h, wn, wl)


def _tc_layer2(aggp, bp, cur1, h, wn, wl, wtg, btg):
    def body(a_ref, b_ref, c_ref, h_ref, wn_ref, wl_ref, wtg_ref, btg_ref,
             o_ref):
        a = a_ref[0] + a_ref[1] + b_ref[0] + b_ref[1]
        cur1 = c_ref[...]
        hh = h_ref[...]
        pre = (jnp.dot(a, wn_ref[...], preferred_element_type=jnp.float32)
               + jnp.dot(cur1, wl_ref[...],
                         preferred_element_type=jnp.float32))
        cur2 = jnp.where(pre >= 0, pre, SLOPE * pre)
        cur2 = _norm_rows(cur2)
        tw = jax.nn.sigmoid(
            jnp.dot(hh, wtg_ref[...], preferred_element_type=jnp.float32)
            + btg_ref[...])
        o_ref[...] = _norm_rows(tw * cur2 + (1.0 - tw) * hh)

    return pl.pallas_call(
        body,
        grid=(N // BN,),
        in_specs=[
            pl.BlockSpec((NC, BN, H), lambda i: (0, i, 0)),
            pl.BlockSpec((NC, BN, H), lambda i: (0, i, 0)),
            pl.BlockSpec((BN, H), lambda i: (i, 0)),
            pl.BlockSpec((BN, H), lambda i: (i, 0)),
            pl.BlockSpec((H, H), lambda i: (0, 0)),
            pl.BlockSpec((H, H), lambda i: (0, 0)),
            pl.BlockSpec((H, H), lambda i: (0, 0)),
            pl.BlockSpec((1, H), lambda i: (0, 0)),
        ],
        out_specs=pl.BlockSpec((BN, H), lambda i: (i, 0)),
        out_shape=jax.ShapeDtypeStruct((N, H), jnp.float32),
    )(aggp, bp, cur1, h, wn, wl, wtg, btg)


def _tc_gru(sp, cntp, emb, h0, wih, whh, bih, bhh):
    def body(s_ref, cnt_ref, e_ref, h0_ref, wih_ref, whh_ref, bih_ref,
             bhh_ref, o_ref):
        ssum = s_ref[0] + s_ref[1]
        ce = cnt_ref[0] + cnt_ref[1]
        emb = e_ref[...]
        cnt = (jnp.sum(ce * emb, axis=1, keepdims=True)
               / jnp.maximum(jnp.sum(emb * emb, axis=1, keepdims=True), 1e-12))
        xm = ssum / jnp.maximum(cnt, 1.0)
        xi = jnp.where(cnt > 0, xm, 0.0)
        xin = jnp.concatenate([e_ref[...], xi], axis=1)
        h0v = h0_ref[...]
        gi = lax.dot_general(xin, wih_ref[...], (((1,), (1,)), ((), ())),
                             preferred_element_type=jnp.float32) + bih_ref[...]
        gh = lax.dot_general(h0v, whh_ref[...], (((1,), (1,)), ((), ())),
                             preferred_element_type=jnp.float32) + bhh_ref[...]
        r = jax.nn.sigmoid(gi[:, :H] + gh[:, :H])
        z = jax.nn.sigmoid(gi[:, H:2 * H] + gh[:, H:2 * H])
        n = jnp.tanh(gi[:, 2 * H:] + r * gh[:, 2 * H:])
        o_ref[...] = _norm_rows((1.0 - z) * n + z * h0v)

    return pl.pallas_call(
        body,
        out_shape=jax.ShapeDtypeStruct((R2P, H), jnp.float32),
    )(sp, cntp, emb, h0, wih, whh, bih, bhh)


def kernel(dynamic_emb, emb_rel, W_neigh1, W_loop1, W_neigh2, W_loop2,
           W_ih, W_hh, b_ih, b_hh, time_gate_weight, time_gate_bias,
           edge_index, edge_type):
    def pad_edges(mode):
        g = GEOM[mode]
        ep = NW * g["cpt"] * g["ch"]
        pad_i = jnp.arange(ep - E, dtype=jnp.int32)
        src2 = jnp.concatenate(
            [edge_index[0], pad_i % N]).reshape(-1, g["ch"])
        dst2 = jnp.concatenate(
            [edge_index[1], N + pad_i % (NP - N)]).reshape(-1, g["ch"])
        et2 = jnp.concatenate(
            [edge_type, R2 + pad_i % (R2P - R2)]).reshape(-1, g["ch"])
        return src2, dst2, et2

    src2, dst2, et2 = pad_edges("dual")
    src2s, dst2s, _ = pad_edges("single")

    emb_pad = jnp.zeros((R2P, H), jnp.float32).at[:R2].set(emb_rel)
    bp, cntp = _sc_pass("dual")(emb_pad, et2, dst2, et2)
    h = _tc_prep(dynamic_emb)
    h0 = emb_pad
    bih2 = b_ih.reshape(1, 3 * H)
    bhh2 = b_hh.reshape(1, 3 * H)
    btg2 = time_gate_bias.reshape(1, H)

    evolve = []
    for _ in range(T):
        a1p, sp = _sc_pass("dual")(h, src2, dst2, et2)
        cur1 = _tc_layer1(a1p, bp, h, W_neigh1, W_loop1)
        h0 = _tc_gru(sp, cntp, emb_pad, h0, W_ih, W_hh, bih2, bhh2)
        a2p = _sc_pass("single")(cur1, src2s, dst2s)
        h = _tc_layer2(a2p, bp, cur1, h, W_neigh2, W_loop2,
                       time_gate_weight, btg2)
        evolve.append(h)
    return jnp.stack(evolve, axis=0), h0[:R2]


# speedup vs baseline: 1.0008x; 1.0008x over previous
import functools

import jax
import jax.numpy as jnp
from jax import lax
from jax.experimental import pallas as pl
from jax.experimental.pallas import tpu as pltpu
from jax.experimental.pallas import tpu_sc as plsc

N = 10000
NP = 10112
R2 = 460
R2P = 512
H = 128
E = 320000
T = 3
SLOPE = (1.0 / 8.0 + 1.0 / 3.0) / 2.0

NC = 2
NS = 16
NW = NC * NS
GEOM = {
    "single": dict(ch=128, cpt=80, cph=16, nbuf=2),
    "dual": dict(ch=128, cpt=80, cph=16, nbuf=2),
}
RPT = NP // NS
BPT = R2P // NS

def _fill(ref, nrows, ncols, val):
    v = jnp.full((16,), val, jnp.float32)

    def body(i, _):
        for j in range(ncols // 16):
            ref[i, pl.ds(j * 16, 16)] = v
        return 0

    lax.fori_loop(0, nrows, body, 0)


@functools.cache
def _sc_pass(mode):
    g = GEOM[mode]
    CH, CPT, CPH, NBUF = g["ch"], g["cpt"], g["cph"], g["nbuf"]
    NPH = CPT // CPH
    RF, RR = RPT // CH, RPT % CH
    mesh = plsc.VectorSubcoreMesh(core_axis_name="c", subcore_axis_name="s")
    outs = [jax.ShapeDtypeStruct((NC, NP, H), jnp.float32)]
    scratch = [pltpu.VMEM_SHARED((NP, H), jnp.float32)]
    if mode == "dual":
        outs.append(jax.ShapeDtypeStruct((NC, R2P, H), jnp.float32))
        scratch.append(pltpu.VMEM_SHARED((R2P, H), jnp.float32))
    scratch.append(pltpu.VMEM((CPH, CH), jnp.int32))
    scratch.append(pltpu.VMEM((CPH, CH), jnp.int32))
    if mode == "dual":
        scratch.append(pltpu.VMEM((CPH, CH), jnp.int32))
    scratch.append(pltpu.VMEM((NBUF, CH, H), jnp.float32))
    scratch.extend([pltpu.SemaphoreType.DMA] * NBUF)
    scratch.extend([pltpu.SemaphoreType.DMA] * NBUF)
    if mode == "dual":
        scratch.extend([pltpu.SemaphoreType.DMA] * NBUF)

    def body(*refs):
        it = iter(refs)
        tab = next(it)
        g2 = next(it)
        a2 = next(it)
        b2 = next(it) if mode == "dual" else None
        out_a = next(it)
        out_b = next(it) if mode == "dual" else None
        acc_a = next(it)
        acc_b = next(it) if mode == "dual" else None
        gv = next(it)
        av = next(it)
        bv = next(it) if mode == "dual" else None
        rows = next(it)
        gsems = [next(it) for _ in range(NBUF)]
        asems = [next(it) for _ in range(NBUF)]
        bsems = [next(it) for _ in range(NBUF)] if mode == "dual" else None

        c = lax.axis_index("c")
        s = lax.axis_index("s")
        w = c * NS + s
        lo = w * CPT

        _fill(rows.at[0], CH, H, 0.0)
        for k in range(RF):
            pltpu.sync_copy(rows.at[0],
                            acc_a.at[pl.ds(s * RPT + k * CH, CH)])
        pltpu.sync_copy(rows.at[0, pl.ds(0, RR)],
                        acc_a.at[pl.ds(s * RPT + RF * CH, RR)])
        if mode == "dual":
            pltpu.sync_copy(rows.at[0, pl.ds(0, BPT)],
                            acc_b.at[pl.ds(s * BPT, BPT)])
        plsc.subcore_barrier()

        def phase(ph, _):
            base = lo + ph * CPH
            pltpu.sync_copy(g2.at[pl.ds(base, CPH)], gv)
            pltpu.sync_copy(a2.at[pl.ds(base, CPH)], av)
            if mode == "dual":
                pltpu.sync_copy(b2.at[pl.ds(base, CPH)], bv)
            cps = [None] * NBUF
            scs = [None] * NBUF
            bcs = [None] * NBUF
            cps[0] = pltpu.async_copy(tab.at[gv.at[0]], rows.at[0], gsems[0])
            cps[1] = pltpu.async_copy(tab.at[gv.at[1]], rows.at[1], gsems[1])
            for j in range(CPH):
                b = j % NBUF
                cps[b].wait()
                rb = rows.at[b]
                scs[b] = pltpu.async_copy(rb, acc_a.at[av.at[j]], asems[b],
                                          add=True)
                if mode == "dual":
                    bcs[b] = pltpu.async_copy(rb, acc_b.at[bv.at[j]],
                                              bsems[b], add=True)
                k = j + 2
                if k < CPH:
                    bk = k % NBUF
                    if scs[bk] is not None:
                        scs[bk].wait()
                        if mode == "dual":
                            bcs[bk].wait()
                    cps[bk] = pltpu.async_copy(tab.at[gv.at[k]], rows.at[bk],
                                               gsems[bk])
            for j in range(max(0, CPH - NBUF), CPH):
                b = j % NBUF
                scs[b].wait()
                if mode == "dual":
                    bcs[b].wait()
            return 0

        lax.fori_loop(0, NPH, phase, 0)
        plsc.subcore_barrier()

        for k in range(RF):
            sl = pl.ds(s * RPT + k * CH, CH)
            pltpu.sync_copy(acc_a.at[sl], out_a.at[c, sl])
        sl = pl.ds(s * RPT + RF * CH, RR)
        pltpu.sync_copy(acc_a.at[sl], out_a.at[c, sl])
        if mode == "dual":
            sl = pl.ds(s * BPT, BPT)
            pltpu.sync_copy(acc_b.at[sl], out_b.at[c, sl])

    out_type = outs[0] if len(outs) == 1 else tuple(outs)
    return pl.kernel(body, out_type=out_type, mesh=mesh,
                     scratch_types=tuple(scratch))


def _norm_rows(x):
    ss = jnp.sum(x * x, axis=-1, keepdims=True)
    return x / jnp.maximum(jnp.sqrt(ss), 1e-12)


BN = 1000


def _tc_prep(x):
    def body(x_ref, o_ref):
        o_ref[...] = _norm_rows(x_ref[...])

    return pl.pallas_call(
        body,
        grid=(N // BN,),
        in_specs=[pl.BlockSpec((BN, H), lambda i: (i, 0))],
        out_specs=pl.BlockSpec((BN, H), lambda i: (i, 0)),
        out_shape=jax.ShapeDtypeStruct((N, H), jnp.float32),
    )(x)


def _tc_layer1(aggp, bp, h, wn, wl):
    def body(a_ref, b_ref, h_ref, wn_ref, wl_ref, o_ref):
        a = a_ref[0] + a_ref[1] + b_ref[0] + b_ref[1]
        pre = (jnp.dot(a, wn_ref[...], preferred_element_type=jnp.float32)
               + jnp.dot(h_ref[...], wl_ref[...],
                         preferred_element_type=jnp.float32))
        o_ref[...] = jnp.where(pre >= 0, pre, SLOPE * pre)

    return pl.pallas_call(
        body,
        grid=(N // BN,),
        in_specs=[
            pl.BlockSpec((NC, BN, H), lambda i: (0, i, 0)),
            pl.BlockSpec((NC, BN, H), lambda i: (0, i, 0)),
            pl.BlockSpec((BN, H), lambda i: (i, 0)),
            pl.BlockSpec((H, H), lambda i: (0, 0)),
            pl.BlockSpec((H, H), lambda i: (0, 0)),
        ],
        out_specs=pl.BlockSpec((BN, H), lambda i: (i, 0)),
        out_shape=jax.ShapeDtypeStruct((N, H), jnp.float32),
    )(aggp, bp, h, wn, wl)


def _tc_layer2(aggp, bp, cur1, h, wn, wl, wtg, btg):
    def body(a_ref, b_ref, c_ref, h_ref, wn_ref, wl_ref, wtg_ref, btg_ref,
             o_ref):
        a = a_ref[0] + a_ref[1] + b_ref[0] + b_ref[1]
        cur1 = c_ref[...]
        hh = h_ref[...]
        pre = (jnp.dot(a, wn_ref[...], preferred_element_type=jnp.float32)
               + jnp.dot(cur1, wl_ref[...],
                         preferred_element_type=jnp.float32))
        cur2 = jnp.where(pre >= 0, pre, SLOPE * pre)
        cur2 = _norm_rows(cur2)
        tw = jax.nn.sigmoid(
            jnp.dot(hh, wtg_ref[...], preferred_element_type=jnp.float32)
            + btg_ref[...])
        o_ref[...] = _norm_rows(tw * cur2 + (1.0 - tw) * hh)

    return pl.pallas_call(
        body,
        grid=(N // BN,),
        in_specs=[
            pl.BlockSpec((NC, BN, H), lambda i: (0, i, 0)),
            pl.BlockSpec((NC, BN, H), lambda i: (0, i, 0)),
            pl.BlockSpec((BN, H), lambda i: (i, 0)),
            pl.BlockSpec((BN, H), lambda i: (i, 0)),
            pl.BlockSpec((H, H), lambda i: (0, 0)),
            pl.BlockSpec((H, H), lambda i: (0, 0)),
            pl.BlockSpec((H, H), lambda i: (0, 0)),
            pl.BlockSpec((1, H), lambda i: (0, 0)),
        ],
        out_specs=pl.BlockSpec((BN, H), lambda i: (i, 0)),
        out_shape=jax.ShapeDtypeStruct((N, H), jnp.float32),
    )(aggp, bp, cur1, h, wn, wl, wtg, btg)


def _tc_gru(sp, cntp, emb, h0, wih, whh, bih, bhh):
    def body(s_ref, cnt_ref, e_ref, h0_ref, wih_ref, whh_ref, bih_ref,
             bhh_ref, o_ref):
        ssum = s_ref[0] + s_ref[1]
        ce = cnt_ref[0] + cnt_ref[1]
        emb = e_ref[...]
        cnt = (jnp.sum(ce * emb, axis=1, keepdims=True)
               / jnp.maximum(jnp.sum(emb * emb, axis=1, keepdims=True), 1e-12))
        xm = ssum / jnp.maximum(cnt, 1.0)
        xi = jnp.where(cnt > 0, xm, 0.0)
        xin = jnp.concatenate([e_ref[...], xi], axis=1)
        h0v = h0_ref[...]
        gi = lax.dot_general(xin, wih_ref[...], (((1,), (1,)), ((), ())),
                             preferred_element_type=jnp.float32) + bih_ref[...]
        gh = lax.dot_general(h0v, whh_ref[...], (((1,), (1,)), ((), ())),
                             preferred_element_type=jnp.float32) + bhh_ref[...]
        r = jax.nn.sigmoid(gi[:, :H] + gh[:, :H])
        z = jax.nn.sigmoid(gi[:, H:2 * H] + gh[:, H:2 * H])
        n = jnp.tanh(gi[:, 2 * H:] + r * gh[:, 2 * H:])
        o_ref[...] = _norm_rows((1.0 - z) * n + z * h0v)

    return pl.pallas_call(
        body,
        out_shape=jax.ShapeDtypeStruct((R2P, H), jnp.float32),
    )(sp, cntp, emb, h0, wih, whh, bih, bhh)


def kernel(dynamic_emb, emb_rel, W_neigh1, W_loop1, W_neigh2, W_loop2,
           W_ih, W_hh, b_ih, b_hh, time_gate_weight, time_gate_bias,
           edge_index, edge_type):
    def pad_edges(mode):
        g = GEOM[mode]
        ep = NW * g["cpt"] * g["ch"]
        pad_i = jnp.arange(ep - E, dtype=jnp.int32)
        src2 = jnp.concatenate(
            [edge_index[0], pad_i % N]).reshape(-1, g["ch"])
        dst2 = jnp.concatenate(
            [edge_index[1], N + pad_i % (NP - N)]).reshape(-1, g["ch"])
        et2 = jnp.concatenate(
            [edge_type, R2 + pad_i % (R2P - R2)]).reshape(-1, g["ch"])
        return src2, dst2, et2

    src2, dst2, et2 = pad_edges("dual")
    src2s, dst2s, _ = pad_edges("single")

    emb_pad = jnp.zeros((R2P, H), jnp.float32).at[:R2].set(emb_rel)
    bp, cntp = _sc_pass("dual")(emb_pad, et2, dst2, et2)
    h = _tc_prep(dynamic_emb)
    h0 = emb_pad
    bih2 = b_ih.reshape(1, 3 * H)
    bhh2 = b_hh.reshape(1, 3 * H)
    btg2 = time_gate_bias.reshape(1, H)

    evolve = []
    for _ in range(T):
        a1p, sp = _sc_pass("dual")(h, src2, dst2, et2)
        cur1 = _tc_layer1(a1p, bp, h, W_neigh1, W_loop1)
        a2p = _sc_pass("single")(cur1, src2s, dst2s)
        h0 = _tc_gru(sp, cntp, emb_pad, h0, W_ih, W_hh, bih2, bhh2)
        h = _tc_layer2(a2p, bp, cur1, h, W_neigh2, W_loop2,
                       time_gate_weight, btg2)
        evolve.append(h)
    return jnp.stack(evolve, axis=0), h0[:R2]


# speedup vs baseline: 1.0181x; 1.0173x over previous
import functools

import jax
import jax.numpy as jnp
from jax import lax
from jax.experimental import pallas as pl
from jax.experimental.pallas import tpu as pltpu
from jax.experimental.pallas import tpu_sc as plsc

N = 10000
NP = 10112
R2 = 460
R2P = 512
H = 128
E = 320000
T = 3
SLOPE = (1.0 / 8.0 + 1.0 / 3.0) / 2.0

NC = 2
NS = 16
NW = NC * NS
GEOM = {
    "single": dict(ch=128, cpt=80, cph=40, nbuf=2),
    "dual": dict(ch=128, cpt=80, cph=16, nbuf=2),
}
RPT = NP // NS
BPT = R2P // NS

def _fill(ref, nrows, ncols, val):
    v = jnp.full((16,), val, jnp.float32)

    def body(i, _):
        for j in range(ncols // 16):
            ref[i, pl.ds(j * 16, 16)] = v
        return 0

    lax.fori_loop(0, nrows, body, 0)


@functools.cache
def _sc_pass(mode):
    g = GEOM[mode]
    CH, CPT, CPH, NBUF = g["ch"], g["cpt"], g["cph"], g["nbuf"]
    NPH = CPT // CPH
    RF, RR = RPT // CH, RPT % CH
    mesh = plsc.VectorSubcoreMesh(core_axis_name="c", subcore_axis_name="s")
    outs = [jax.ShapeDtypeStruct((NC, NP, H), jnp.float32)]
    scratch = [pltpu.VMEM_SHARED((NP, H), jnp.float32)]
    if mode == "dual":
        outs.append(jax.ShapeDtypeStruct((NC, R2P, H), jnp.float32))
        scratch.append(pltpu.VMEM_SHARED((R2P, H), jnp.float32))
    scratch.append(pltpu.VMEM((CPH, CH), jnp.int32))
    scratch.append(pltpu.VMEM((CPH, CH), jnp.int32))
    if mode == "dual":
        scratch.append(pltpu.VMEM((CPH, CH), jnp.int32))
    scratch.append(pltpu.VMEM((NBUF, CH, H), jnp.float32))
    scratch.extend([pltpu.SemaphoreType.DMA] * NBUF)
    scratch.extend([pltpu.SemaphoreType.DMA] * NBUF)
    if mode == "dual":
        scratch.extend([pltpu.SemaphoreType.DMA] * NBUF)

    def body(*refs):
        it = iter(refs)
        tab = next(it)
        g2 = next(it)
        a2 = next(it)
        b2 = next(it) if mode == "dual" else None
        out_a = next(it)
        out_b = next(it) if mode == "dual" else None
        acc_a = next(it)
        acc_b = next(it) if mode == "dual" else None
        gv = next(it)
        av = next(it)
        bv = next(it) if mode == "dual" else None
        rows = next(it)
        gsems = [next(it) for _ in range(NBUF)]
        asems = [next(it) for _ in range(NBUF)]
        bsems = [next(it) for _ in range(NBUF)] if mode == "dual" else None

        c = lax.axis_index("c")
        s = lax.axis_index("s")
        w = c * NS + s
        lo = w * CPT

        _fill(rows.at[0], CH, H, 0.0)
        for k in range(RF):
            pltpu.sync_copy(rows.at[0],
                            acc_a.at[pl.ds(s * RPT + k * CH, CH)])
        pltpu.sync_copy(rows.at[0, pl.ds(0, RR)],
                        acc_a.at[pl.ds(s * RPT + RF * CH, RR)])
        if mode == "dual":
            pltpu.sync_copy(rows.at[0, pl.ds(0, BPT)],
                            acc_b.at[pl.ds(s * BPT, BPT)])
        plsc.subcore_barrier()

        def phase(ph, _):
            base = lo + ph * CPH
            pltpu.sync_copy(g2.at[pl.ds(base, CPH)], gv)
            pltpu.sync_copy(a2.at[pl.ds(base, CPH)], av)
            if mode == "dual":
                pltpu.sync_copy(b2.at[pl.ds(base, CPH)], bv)
            cps = [None] * NBUF
            scs = [None] * NBUF
            bcs = [None] * NBUF
            cps[0] = pltpu.async_copy(tab.at[gv.at[0]], rows.at[0], gsems[0])
            cps[1] = pltpu.async_copy(tab.at[gv.at[1]], rows.at[1], gsems[1])
            for j in range(CPH):
                b = j % NBUF
                cps[b].wait()
                rb = rows.at[b]
                scs[b] = pltpu.async_copy(rb, acc_a.at[av.at[j]], asems[b],
                                          add=True)
                if mode == "dual":
                    bcs[b] = pltpu.async_copy(rb, acc_b.at[bv.at[j]],
                                              bsems[b], add=True)
                k = j + 2
                if k < CPH:
                    bk = k % NBUF
                    if scs[bk] is not None:
                        scs[bk].wait()
                        if mode == "dual":
                            bcs[bk].wait()
                    cps[bk] = pltpu.async_copy(tab.at[gv.at[k]], rows.at[bk],
                                               gsems[bk])
            for j in range(max(0, CPH - NBUF), CPH):
                b = j % NBUF
                scs[b].wait()
                if mode == "dual":
                    bcs[b].wait()
            return 0

        lax.fori_loop(0, NPH, phase, 0)
        plsc.subcore_barrier()

        for k in range(RF):
            sl = pl.ds(s * RPT + k * CH, CH)
            pltpu.sync_copy(acc_a.at[sl], out_a.at[c, sl])
        sl = pl.ds(s * RPT + RF * CH, RR)
        pltpu.sync_copy(acc_a.at[sl], out_a.at[c, sl])
        if mode == "dual":
            sl = pl.ds(s * BPT, BPT)
            pltpu.sync_copy(acc_b.at[sl], out_b.at[c, sl])

    out_type = outs[0] if len(outs) == 1 else tuple(outs)
    return pl.kernel(body, out_type=out_type, mesh=mesh,
                     scratch_types=tuple(scratch))


def _norm_rows(x):
    ss = jnp.sum(x * x, axis=-1, keepdims=True)
    return x / jnp.maximum(jnp.sqrt(ss), 1e-12)


BN = 1000


def _tc_prep(x):
    def body(x_ref, o_ref):
        o_ref[...] = _norm_rows(x_ref[...])

    return pl.pallas_call(
        body,
        grid=(N // BN,),
        in_specs=[pl.BlockSpec((BN, H), lambda i: (i, 0))],
        out_specs=pl.BlockSpec((BN, H), lambda i: (i, 0)),
        out_shape=jax.ShapeDtypeStruct((N, H), jnp.float32),
    )(x)


def _tc_layer1(aggp, bp, h, wn, wl):
    def body(a_ref, b_ref, h_ref, wn_ref, wl_ref, o_ref):
        a = a_ref[0] + a_ref[1] + b_ref[0] + b_ref[1]
        pre = (jnp.dot(a, wn_ref[...], preferred_element_type=jnp.float32)
               + jnp.dot(h_ref[...], wl_ref[...],
                         preferred_element_type=jnp.float32))
        o_ref[...] = jnp.where(pre >= 0, pre, SLOPE * pre)

    return pl.pallas_call(
        body,
        grid=(N // BN,),
        in_specs=[
            pl.BlockSpec((NC, BN, H), lambda i: (0, i, 0)),
            pl.BlockSpec((NC, BN, H), lambda i: (0, i, 0)),
            pl.BlockSpec((BN, H), lambda i: (i, 0)),
            pl.BlockSpec((H, H), lambda i: (0, 0)),
            pl.BlockSpec((H, H), lambda i: (0, 0)),
        ],
        out_specs=pl.BlockSpec((BN, H), lambda i: (i, 0)),
        out_shape=jax.ShapeDtypeStruct((N, H), jnp.float32),
    )(aggp, bp, h, wn, wl)


def _tc_layer2(aggp, bp, cur1, h, wn, wl, wtg, btg):
    def body(a_ref, b_ref, c_ref, h_ref, wn_ref, wl_ref, wtg_ref, btg_ref,
             o_ref):
        a = a_ref[0] + a_ref[1] + b_ref[0] + b_ref[1]
        cur1 = c_ref[...]
        hh = h_ref[...]
        pre = (jnp.dot(a, wn_ref[...], preferred_element_type=jnp.float32)
               + jnp.dot(cur1, wl_ref[...],
                         preferred_element_type=jnp.float32))
        cur2 = jnp.where(pre >= 0, pre, SLOPE * pre)
        cur2 = _norm_rows(cur2)
        tw = jax.nn.sigmoid(
            jnp.dot(hh, wtg_ref[...], preferred_element_type=jnp.float32)
            + btg_ref[...])
        o_ref[...] = _norm_rows(tw * cur2 + (1.0 - tw) * hh)

    return pl.pallas_call(
        body,
        grid=(N // BN,),
        in_specs=[
            pl.BlockSpec((NC, BN, H), lambda i: (0, i, 0)),
            pl.BlockSpec((NC, BN, H), lambda i: (0, i, 0)),
            pl.BlockSpec((BN, H), lambda i: (i, 0)),
            pl.BlockSpec((BN, H), lambda i: (i, 0)),
            pl.BlockSpec((H, H), lambda i: (0, 0)),
            pl.BlockSpec((H, H), lambda i: (0, 0)),
            pl.BlockSpec((H, H), lambda i: (0, 0)),
            pl.BlockSpec((1, H), lambda i: (0, 0)),
        ],
        out_specs=pl.BlockSpec((BN, H), lambda i: (i, 0)),
        out_shape=jax.ShapeDtypeStruct((N, H), jnp.float32),
    )(aggp, bp, cur1, h, wn, wl, wtg, btg)


def _tc_gru(sp, cntp, emb, h0, wih, whh, bih, bhh):
    def body(s_ref, cnt_ref, e_ref, h0_ref, wih_ref, whh_ref, bih_ref,
             bhh_ref, o_ref):
        ssum = s_ref[0] + s_ref[1]
        ce = cnt_ref[0] + cnt_ref[1]
        emb = e_ref[...]
        cnt = (jnp.sum(ce * emb, axis=1, keepdims=True)
               / jnp.maximum(jnp.sum(emb * emb, axis=1, keepdims=True), 1e-12))
        xm = ssum / jnp.maximum(cnt, 1.0)
        xi = jnp.where(cnt > 0, xm, 0.0)
        xin = jnp.concatenate([e_ref[...], xi], axis=1)
        h0v = h0_ref[...]
        gi = lax.dot_general(xin, wih_ref[...], (((1,), (1,)), ((), ())),
                             preferred_element_type=jnp.float32) + bih_ref[...]
        gh = lax.dot_general(h0v, whh_ref[...], (((1,), (1,)), ((), ())),
                             preferred_element_type=jnp.float32) + bhh_ref[...]
        r = jax.nn.sigmoid(gi[:, :H] + gh[:, :H])
        z = jax.nn.sigmoid(gi[:, H:2 * H] + gh[:, H:2 * H])
        n = jnp.tanh(gi[:, 2 * H:] + r * gh[:, 2 * H:])
        o_ref[...] = _norm_rows((1.0 - z) * n + z * h0v)

    return pl.pallas_call(
        body,
        out_shape=jax.ShapeDtypeStruct((R2P, H), jnp.float32),
    )(sp, cntp, emb, h0, wih, whh, bih, bhh)


def kernel(dynamic_emb, emb_rel, W_neigh1, W_loop1, W_neigh2, W_loop2,
           W_ih, W_hh, b_ih, b_hh, time_gate_weight, time_gate_bias,
           edge_index, edge_type):
    def pad_edges(mode):
        g = GEOM[mode]
        ep = NW * g["cpt"] * g["ch"]
        pad_i = jnp.arange(ep - E, dtype=jnp.int32)
        src2 = jnp.concatenate(
            [edge_index[0], pad_i % N]).reshape(-1, g["ch"])
        dst2 = jnp.concatenate(
            [edge_index[1], N + pad_i % (NP - N)]).reshape(-1, g["ch"])
        et2 = jnp.concatenate(
            [edge_type, R2 + pad_i % (R2P - R2)]).reshape(-1, g["ch"])
        return src2, dst2, et2

    src2, dst2, et2 = pad_edges("dual")
    src2s, dst2s, _ = pad_edges("single")

    emb_pad = jnp.zeros((R2P, H), jnp.float32).at[:R2].set(emb_rel)
    bp, cntp = _sc_pass("dual")(emb_pad, et2, dst2, et2)
    h = _tc_prep(dynamic_emb)
    h0 = emb_pad
    bih2 = b_ih.reshape(1, 3 * H)
    bhh2 = b_hh.reshape(1, 3 * H)
    btg2 = time_gate_bias.reshape(1, H)

    evolve = []
    for _ in range(T):
        a1p, sp = _sc_pass("dual")(h, src2, dst2, et2)
        cur1 = _tc_layer1(a1p, bp, h, W_neigh1, W_loop1)
        a2p = _sc_pass("single")(cur1, src2s, dst2s)
        h0 = _tc_gru(sp, cntp, emb_pad, h0, W_ih, W_hh, bih2, bhh2)
        h = _tc_layer2(a2p, bp, cur1, h, W_neigh2, W_loop2,
                       time_gate_weight, btg2)
        evolve.append(h)
    return jnp.stack(evolve, axis=0), h0[:R2]
